# baseline (speedup 1.0000x reference)
import jax
import jax.numpy as jnp
from jax.experimental import pallas as pl


def _count_body(x_ref, out_ref):
    blk = x_ref[...]
    out_ref[0, :, :] = jnp.sum((blk == blk).astype(jnp.int32), axis=0,
                               keepdims=True)


def kernel(x):
    n = x.shape[0]
    xb = x.reshape(n // 128, 128)
    counts = pl.pallas_call(
        _count_body,
        grid=(8,),
        in_specs=[pl.BlockSpec((n // 128 // 8, 128), lambda i: (i, 0))],
        out_specs=pl.BlockSpec((1, 1, 128), lambda i: (i, 0, 0)),
        out_shape=jax.ShapeDtypeStruct((8, 1, 128), jnp.int32),
    )(xb)
    return jnp.sum(counts) == n
